# baseline (speedup 1.0000x reference)
import functools

import jax
import jax.numpy as jnp
from jax import lax
from jax.experimental import pallas as pl
from jax.experimental.pallas import tpu as pltpu
from jax.experimental.pallas import tpu_sc as plsc

_NC = 2
_NS = 16
_NW = _NC * _NS
_C = 128
_BLK = 8
_NB = 1000
_CG = 8


def _deg_body(dst3, ew3, zeros1, degp, dstv, ewv, degv, *, nchunk):
    c = lax.axis_index("c")
    s = lax.axis_index("s")
    wid = c * _NS + s
    pltpu.sync_copy(zeros1, degv)

    def blk(b, carry):
        pltpu.sync_copy(dst3.at[wid, b], dstv)
        pltpu.sync_copy(ew3.at[wid, b], ewv)

        def chunk(j, c2):
            for g in range(_C // 16):
                d16 = dstv[j, pl.ds(g * 16, 16)]
                w16 = ewv[j, pl.ds(g * 16, 16)]
                plsc.addupdate_scatter(degv, [d16], w16)
            return c2

        lax.fori_loop(0, _BLK, chunk, 0)
        return carry

    lax.fori_loop(0, nchunk // _BLK, blk, 0)
    pltpu.sync_copy(degv, degp.at[wid])


_ND = 4


def _spmm_body(xcol, src3, dst3, ew3, S, srcv, dstv, ewv, idxbs, rowss, acc,
               gsems, *, nchunk, n_nodes, n_pad):
    c = lax.axis_index("c")
    s = lax.axis_index("s")
    zv = jnp.zeros((16,), jnp.float32)

    def zloop(i, carry):
        for u in range(8):
            acc[pl.ds(i * 128 + u * 16, 16)] = zv
        return carry

    lax.fori_loop(0, n_pad * _CG // 128, zloop, 0)

    soff = jnp.full((16,), s * n_nodes, jnp.int32)
    iota = lax.iota(jnp.int32, 16)
    half = iota // 8
    colofs = iota - half * 8

    def stage_idx(j, k):
        for u in range(_C // 16):
            sl = pl.ds(u * 16, 16)
            idxbs[k][sl] = srcv[j, sl] + soff

    def wblk(t, carry):
        w = c * _NS + t // (nchunk // _BLK)
        b = t % (nchunk // _BLK)
        pltpu.sync_copy(src3.at[w, b], srcv)
        pltpu.sync_copy(dst3.at[w, b], dstv)
        pltpu.sync_copy(ew3.at[w, b], ewv)

        for k in range(_ND):
            stage_idx(k, k)
            pltpu.async_copy(xcol.at[idxbs[k]], rowss[k], gsems[k])
        for j in range(_BLK):
            k = j % _ND
            pltpu.make_async_copy(xcol.at[idxbs[k]], rowss[k],
                                  gsems[k]).wait()
            jvec = jnp.full((16,), j, jnp.int32)
            for p in range(_C // 2):
                pidx = half + 2 * p
                d2 = plsc.load_gather(dstv, [jvec, pidx])
                w2 = plsc.load_gather(ewv, [jvec, pidx])
                val = plsc.load_gather(rowss[k], [pidx, colofs]) * w2
                addr = d2 * _CG + colofs
                plsc.addupdate_scatter(acc, [addr], val)
            if j + _ND < _BLK:
                stage_idx(j + _ND, k)
                pltpu.async_copy(xcol.at[idxbs[k]], rowss[k], gsems[k])
        return carry

    lax.fori_loop(0, _NS * (nchunk // _BLK), wblk, 0)
    wid = c * _NS + s
    pltpu.sync_copy(acc, S.at[pl.ds(wid * n_pad * _CG, n_pad * _CG)])


def _xws_body(x_ref, w_ref, degp_ref, xcol_ref, *, hdim):
    deg = 1.0 + jnp.sum(degp_ref[...], axis=1)
    dis = lax.rsqrt(deg)
    xw = lax.dot_general(x_ref[...], w_ref[...], (((1,), (1,)), ((), ())),
                         preferred_element_type=jnp.float32)
    xws = xw * dis[:, None]
    nb = xws.shape[0]
    xcol_ref[...] = xws.reshape(nb, hdim // _CG, _CG).transpose(1, 0, 2)


def _gru_body(s_ref, xcol_ref, degp_ref, x_ref, h_ref, wihx_ref, wihg_ref,
              whh_ref, bih_ref, bhh_ref, bgcn_ref, out_ref, *, hdim):
    deg = 1.0 + jnp.sum(degp_ref[...], axis=1)
    dis = lax.rsqrt(deg)[:, None]
    nb = x_ref.shape[0]
    ssum = (s_ref[0] + s_ref[1]).transpose(1, 0, 2).reshape(nb, hdim)
    xws = xcol_ref[...].transpose(1, 0, 2).reshape(nb, hdim)
    g = jax.nn.sigmoid(dis * (ssum + xws) + bgcn_ref[...])
    dn = (((1,), (1,)), ((), ()))
    gi = (lax.dot_general(x_ref[...], wihx_ref[...], dn,
                          preferred_element_type=jnp.float32)
          + lax.dot_general(g, wihg_ref[...], dn,
                            preferred_element_type=jnp.float32)
          + bih_ref[...])
    gh = (lax.dot_general(h_ref[...], whh_ref[...], dn,
                          preferred_element_type=jnp.float32)
          + bhh_ref[...])
    r = jax.nn.sigmoid(gi[:, :hdim] + gh[:, :hdim])
    z = jax.nn.sigmoid(gi[:, hdim:2 * hdim] + gh[:, hdim:2 * hdim])
    n = jnp.tanh(gi[:, 2 * hdim:] + r * gh[:, 2 * hdim:])
    out_ref[...] = (1.0 - z) * n + z * h_ref[...]


def kernel(x, edge_index, edge_weight, h, W_gcn, b_gcn, W_ih, W_hh, b_ih,
           b_hh):
    n_nodes, d_in = x.shape
    hdim = h.shape[1]
    n_edges = edge_weight.shape[0]
    e_grp = _NW * _C * _BLK
    e_cap = ((n_edges + e_grp - 1) // e_grp) * e_grp
    nchunk = e_cap // (_NW * _C)
    n_pad = ((n_nodes + 64 * _NS - 1) // (64 * _NS)) * (64 * _NS)

    src = edge_index[0].astype(jnp.int32)
    dst = edge_index[1].astype(jnp.int32)
    ew = edge_weight.astype(jnp.float32)
    if e_cap != n_edges:
        pad = e_cap - n_edges
        src = jnp.concatenate([src, jnp.zeros((pad,), jnp.int32)])
        dst = jnp.concatenate([dst, jnp.zeros((pad,), jnp.int32)])
        ew = jnp.concatenate([ew, jnp.zeros((pad,), jnp.float32)])
    nblk = nchunk // _BLK
    src3 = src.reshape(_NW, nblk, _BLK, _C)
    dst3 = dst.reshape(_NW, nblk, _BLK, _C)
    ew3 = ew.reshape(_NW, nblk, _BLK, _C)
    zeros1 = jnp.zeros((n_pad,), jnp.float32)

    mesh = plsc.VectorSubcoreMesh(core_axis_name="c", subcore_axis_name="s")
    sc_params = pltpu.CompilerParams(needs_layout_passes=False,
                                     use_tc_tiling_on_sc=False)

    degp = pl.kernel(
        functools.partial(_deg_body, nchunk=nchunk),
        out_type=jax.ShapeDtypeStruct((_NW, n_pad), jnp.float32),
        mesh=mesh,
        scratch_types=[
            pltpu.VMEM((_BLK, _C), jnp.int32),
            pltpu.VMEM((_BLK, _C), jnp.float32),
            pltpu.VMEM((n_pad,), jnp.float32),
        ],
        compiler_params=sc_params,
    )(dst3, ew3, zeros1)
    degp_t = degp.T[:n_nodes]

    grid = n_nodes // _NB
    xcol = pl.pallas_call(
        functools.partial(_xws_body, hdim=hdim),
        grid=(grid,),
        in_specs=[
            pl.BlockSpec((_NB, d_in), lambda i: (i, 0)),
            pl.BlockSpec((hdim, d_in), lambda i: (0, 0)),
            pl.BlockSpec((_NB, _NW), lambda i: (i, 0)),
        ],
        out_specs=pl.BlockSpec((hdim // _CG, _NB, _CG), lambda i: (0, i, 0)),
        out_shape=jax.ShapeDtypeStruct((hdim // _CG, n_nodes, _CG),
                                       jnp.float32),
    )(x, W_gcn, degp_t)
    xcol_flat = xcol.reshape(hdim // _CG * n_nodes, _CG)

    S = pl.kernel(
        functools.partial(_spmm_body, nchunk=nchunk, n_nodes=n_nodes,
                          n_pad=n_pad),
        out_type=jax.ShapeDtypeStruct((_NW * n_pad * _CG,), jnp.float32),
        mesh=mesh,
        scratch_types=[
            pltpu.VMEM((_BLK, _C), jnp.int32),
            pltpu.VMEM((_BLK, _C), jnp.int32),
            pltpu.VMEM((_BLK, _C), jnp.float32),
            [pltpu.VMEM((_C,), jnp.int32) for _ in range(_ND)],
            [pltpu.VMEM((_C, _CG), jnp.float32) for _ in range(_ND)],
            pltpu.VMEM((n_pad * _CG,), jnp.float32),
            [pltpu.SemaphoreType.DMA for _ in range(_ND)],
        ],
        compiler_params=sc_params,
    )(xcol_flat, src3, dst3, ew3)
    S4 = S.reshape(_NC, _NS, n_pad, _CG)[:, :, :n_nodes, :]

    nb2 = 400
    out = pl.pallas_call(
        functools.partial(_gru_body, hdim=hdim),
        grid=(n_nodes // nb2,),
        in_specs=[
            pl.BlockSpec((_NC, _NS, nb2, _CG), lambda i: (0, 0, i, 0)),
            pl.BlockSpec((hdim // _CG, nb2, _CG), lambda i: (0, i, 0)),
            pl.BlockSpec((nb2, _NW), lambda i: (i, 0)),
            pl.BlockSpec((nb2, d_in), lambda i: (i, 0)),
            pl.BlockSpec((nb2, hdim), lambda i: (i, 0)),
            pl.BlockSpec((3 * hdim, d_in), lambda i: (0, 0)),
            pl.BlockSpec((3 * hdim, hdim), lambda i: (0, 0)),
            pl.BlockSpec((3 * hdim, hdim), lambda i: (0, 0)),
            pl.BlockSpec((1, 3 * hdim), lambda i: (0, 0)),
            pl.BlockSpec((1, 3 * hdim), lambda i: (0, 0)),
            pl.BlockSpec((1, hdim), lambda i: (0, 0)),
        ],
        out_specs=pl.BlockSpec((nb2, hdim), lambda i: (i, 0)),
        out_shape=jax.ShapeDtypeStruct((n_nodes, hdim), jnp.float32),
    )(S4, xcol, degp_t, x, h, W_ih[:, :d_in], W_ih[:, d_in:], W_hh,
      b_ih.reshape(1, -1), b_hh.reshape(1, -1), b_gcn.reshape(1, -1))
    return out


# speedup vs baseline: 1.2114x; 1.2114x over previous
import functools

import jax
import jax.numpy as jnp
from jax import lax
from jax.experimental import pallas as pl
from jax.experimental.pallas import tpu as pltpu
from jax.experimental.pallas import tpu_sc as plsc

_NC = 2
_NS = 16
_NW = _NC * _NS
_C = 128
_BLK = 8
_NB = 1000
_CG = 8


def _deg_body(dst3, ew3, zeros1, degp, dstv, ewv, degv, *, nchunk):
    c = lax.axis_index("c")
    s = lax.axis_index("s")
    wid = c * _NS + s
    pltpu.sync_copy(zeros1, degv)

    def blk(b, carry):
        pltpu.sync_copy(dst3.at[wid, b], dstv)
        pltpu.sync_copy(ew3.at[wid, b], ewv)

        def chunk(j, c2):
            for g in range(_C // 16):
                d16 = dstv[j, pl.ds(g * 16, 16)]
                w16 = ewv[j, pl.ds(g * 16, 16)]
                plsc.addupdate_scatter(degv, [d16], w16)
            return c2

        lax.fori_loop(0, _BLK, chunk, 0)
        return carry

    lax.fori_loop(0, nchunk // _BLK, blk, 0)
    pltpu.sync_copy(degv, degp.at[wid])


_ND = 4


def _spmm_body(xcol, src3, ax4, ewx4, S, srcv, axv, ewxv, idxbs, rowss, acc,
               gsems, *, nchunk, n_nodes, n_pad):
    c = lax.axis_index("c")
    s = lax.axis_index("s")
    zv = jnp.zeros((16,), jnp.float32)

    def zloop(i, carry):
        for u in range(8):
            acc[pl.ds(i * 128 + u * 16, 16)] = zv
        return carry

    lax.fori_loop(0, n_pad * _CG // 128, zloop, 0)

    soff = jnp.full((16,), s * n_nodes, jnp.int32)
    iota = lax.iota(jnp.int32, 16)
    half = iota // 8
    colofs = iota - half * 8

    def stage_idx(j, k):
        for u in range(_C // 16):
            sl = pl.ds(u * 16, 16)
            idxbs[k][sl] = srcv[j, sl] + soff

    def wblk(t, carry):
        w = c * _NS + t // (nchunk // _BLK)
        b = t % (nchunk // _BLK)
        pltpu.sync_copy(src3.at[w, b], srcv)
        pltpu.sync_copy(ax4.at[w, b], axv)
        pltpu.sync_copy(ewx4.at[w, b], ewxv)

        for k in range(_ND):
            stage_idx(k, k)
            pltpu.async_copy(xcol.at[idxbs[k]], rowss[k], gsems[k])
        for j in range(_BLK):
            k = j % _ND
            pltpu.make_async_copy(xcol.at[idxbs[k]], rowss[k],
                                  gsems[k]).wait()
            for p in range(_C // 2):
                sl = pl.ds(p * 16, 16)
                w2 = ewxv[j, sl]
                addr = axv[j, sl]
                pidx = half + 2 * p
                val = plsc.load_gather(rowss[k], [pidx, colofs]) * w2
                plsc.addupdate_scatter(acc, [addr], val)
            if j + _ND < _BLK:
                stage_idx(j + _ND, k)
                pltpu.async_copy(xcol.at[idxbs[k]], rowss[k], gsems[k])
        return carry

    lax.fori_loop(0, _NS * (nchunk // _BLK), wblk, 0)
    wid = c * _NS + s
    pltpu.sync_copy(acc, S.at[pl.ds(wid * n_pad * _CG, n_pad * _CG)])


def _xws_body(x_ref, w_ref, degp_ref, xcol_ref, *, hdim):
    deg = 1.0 + jnp.sum(degp_ref[...], axis=1)
    dis = lax.rsqrt(deg)
    xw = lax.dot_general(x_ref[...], w_ref[...], (((1,), (1,)), ((), ())),
                         preferred_element_type=jnp.float32)
    xws = xw * dis[:, None]
    nb = xws.shape[0]
    xcol_ref[...] = xws.reshape(nb, hdim // _CG, _CG).transpose(1, 0, 2)


def _gru_body(s_ref, xcol_ref, degp_ref, x_ref, h_ref, wihx_ref, wihg_ref,
              whh_ref, bih_ref, bhh_ref, bgcn_ref, out_ref, *, hdim):
    deg = 1.0 + jnp.sum(degp_ref[...], axis=1)
    dis = lax.rsqrt(deg)[:, None]
    nb = x_ref.shape[0]
    ssum = (s_ref[0] + s_ref[1]).transpose(1, 0, 2).reshape(nb, hdim)
    xws = xcol_ref[...].transpose(1, 0, 2).reshape(nb, hdim)
    g = jax.nn.sigmoid(dis * (ssum + xws) + bgcn_ref[...])
    dn = (((1,), (1,)), ((), ()))
    gi = (lax.dot_general(x_ref[...], wihx_ref[...], dn,
                          preferred_element_type=jnp.float32)
          + lax.dot_general(g, wihg_ref[...], dn,
                            preferred_element_type=jnp.float32)
          + bih_ref[...])
    gh = (lax.dot_general(h_ref[...], whh_ref[...], dn,
                          preferred_element_type=jnp.float32)
          + bhh_ref[...])
    r = jax.nn.sigmoid(gi[:, :hdim] + gh[:, :hdim])
    z = jax.nn.sigmoid(gi[:, hdim:2 * hdim] + gh[:, hdim:2 * hdim])
    n = jnp.tanh(gi[:, 2 * hdim:] + r * gh[:, 2 * hdim:])
    out_ref[...] = (1.0 - z) * n + z * h_ref[...]


def kernel(x, edge_index, edge_weight, h, W_gcn, b_gcn, W_ih, W_hh, b_ih,
           b_hh):
    n_nodes, d_in = x.shape
    hdim = h.shape[1]
    n_edges = edge_weight.shape[0]
    e_grp = _NW * _C * _BLK
    e_cap = ((n_edges + e_grp - 1) // e_grp) * e_grp
    nchunk = e_cap // (_NW * _C)
    n_pad = ((n_nodes + 64 * _NS - 1) // (64 * _NS)) * (64 * _NS)

    src = edge_index[0].astype(jnp.int32)
    dst = edge_index[1].astype(jnp.int32)
    ew = edge_weight.astype(jnp.float32)
    if e_cap != n_edges:
        pad = e_cap - n_edges
        src = jnp.concatenate([src, jnp.zeros((pad,), jnp.int32)])
        dst = jnp.concatenate([dst, jnp.zeros((pad,), jnp.int32)])
        ew = jnp.concatenate([ew, jnp.zeros((pad,), jnp.float32)])
    nblk = nchunk // _BLK
    src3 = src.reshape(_NW, nblk, _BLK, _C)
    dst3 = dst.reshape(_NW, nblk, _BLK, _C)
    ew3 = ew.reshape(_NW, nblk, _BLK, _C)
    ax4 = (dst[:, None] * _CG + jnp.arange(_CG, dtype=jnp.int32)).reshape(
        _NW, nblk, _BLK, _C * _CG)
    ewx4 = jnp.broadcast_to(ew[:, None], (e_cap, _CG)).reshape(
        _NW, nblk, _BLK, _C * _CG)
    zeros1 = jnp.zeros((n_pad,), jnp.float32)

    mesh = plsc.VectorSubcoreMesh(core_axis_name="c", subcore_axis_name="s")
    sc_params = pltpu.CompilerParams(needs_layout_passes=False,
                                     use_tc_tiling_on_sc=False)

    degp = pl.kernel(
        functools.partial(_deg_body, nchunk=nchunk),
        out_type=jax.ShapeDtypeStruct((_NW, n_pad), jnp.float32),
        mesh=mesh,
        scratch_types=[
            pltpu.VMEM((_BLK, _C), jnp.int32),
            pltpu.VMEM((_BLK, _C), jnp.float32),
            pltpu.VMEM((n_pad,), jnp.float32),
        ],
        compiler_params=sc_params,
    )(dst3, ew3, zeros1)
    degp_t = degp.T[:n_nodes]

    grid = n_nodes // _NB
    xcol = pl.pallas_call(
        functools.partial(_xws_body, hdim=hdim),
        grid=(grid,),
        in_specs=[
            pl.BlockSpec((_NB, d_in), lambda i: (i, 0)),
            pl.BlockSpec((hdim, d_in), lambda i: (0, 0)),
            pl.BlockSpec((_NB, _NW), lambda i: (i, 0)),
        ],
        out_specs=pl.BlockSpec((hdim // _CG, _NB, _CG), lambda i: (0, i, 0)),
        out_shape=jax.ShapeDtypeStruct((hdim // _CG, n_nodes, _CG),
                                       jnp.float32),
    )(x, W_gcn, degp_t)
    xcol_flat = xcol.reshape(hdim // _CG * n_nodes, _CG)

    S = pl.kernel(
        functools.partial(_spmm_body, nchunk=nchunk, n_nodes=n_nodes,
                          n_pad=n_pad),
        out_type=jax.ShapeDtypeStruct((_NW * n_pad * _CG,), jnp.float32),
        mesh=mesh,
        scratch_types=[
            pltpu.VMEM((_BLK, _C), jnp.int32),
            pltpu.VMEM((_BLK, _C * _CG), jnp.int32),
            pltpu.VMEM((_BLK, _C * _CG), jnp.float32),
            [pltpu.VMEM((_C,), jnp.int32) for _ in range(_ND)],
            [pltpu.VMEM((_C, _CG), jnp.float32) for _ in range(_ND)],
            pltpu.VMEM((n_pad * _CG,), jnp.float32),
            [pltpu.SemaphoreType.DMA for _ in range(_ND)],
        ],
        compiler_params=sc_params,
    )(xcol_flat, src3, ax4, ewx4)
    S4 = S.reshape(_NC, _NS, n_pad, _CG)[:, :, :n_nodes, :]

    nb2 = 400
    out = pl.pallas_call(
        functools.partial(_gru_body, hdim=hdim),
        grid=(n_nodes // nb2,),
        in_specs=[
            pl.BlockSpec((_NC, _NS, nb2, _CG), lambda i: (0, 0, i, 0)),
            pl.BlockSpec((hdim // _CG, nb2, _CG), lambda i: (0, i, 0)),
            pl.BlockSpec((nb2, _NW), lambda i: (i, 0)),
            pl.BlockSpec((nb2, d_in), lambda i: (i, 0)),
            pl.BlockSpec((nb2, hdim), lambda i: (i, 0)),
            pl.BlockSpec((3 * hdim, d_in), lambda i: (0, 0)),
            pl.BlockSpec((3 * hdim, hdim), lambda i: (0, 0)),
            pl.BlockSpec((3 * hdim, hdim), lambda i: (0, 0)),
            pl.BlockSpec((1, 3 * hdim), lambda i: (0, 0)),
            pl.BlockSpec((1, 3 * hdim), lambda i: (0, 0)),
            pl.BlockSpec((1, hdim), lambda i: (0, 0)),
        ],
        out_specs=pl.BlockSpec((nb2, hdim), lambda i: (i, 0)),
        out_shape=jax.ShapeDtypeStruct((n_nodes, hdim), jnp.float32),
    )(S4, xcol, degp_t, x, h, W_ih[:, :d_in], W_ih[:, d_in:], W_hh,
      b_ih.reshape(1, -1), b_hh.reshape(1, -1), b_gcn.reshape(1, -1))
    return out
